# baseline (speedup 1.0000x reference)
import functools

import jax
import jax.numpy as jnp
from jax.experimental import pallas as pl
from jax.experimental.pallas import tpu as pltpu

TB = 512


def _round_up(x, m):
    return (x + m - 1) // m * m


def _lenet_kernel(x_ref, m1_ref, b1_ref, m2_ref, b3_ref, w5_ref,
                  b5_ref, w6_ref, b6_ref, w7_ref, b7_ref, o_ref):
    tb = o_ref.shape[1]
    xp = x_ref[...]

    xcat = jnp.concatenate([xp[:, j:j + 28, :] for j in range(5)], axis=0)
    c1 = jax.lax.dot_general(m1_ref[...], xcat, (((1,), (0,)), ((), ())),
                             preferred_element_type=jnp.float32)
    c1 = jnp.maximum(c1.reshape(6, 28, 28, tb) + b1_ref[...], 0.0)

    ph = jnp.max(c1.reshape(6, 14, 2, 28, tb), axis=2)
    shw = jnp.concatenate([ph[:, :, 1:, :], ph[:, :, :1, :]], axis=2)
    p1 = jnp.maximum(ph, shw)

    x2 = jnp.concatenate(
        [p1[:, :, 2 * j:2 * j + 20, :].reshape(84, 20, tb) for j in range(5)],
        axis=0)
    c2 = jax.lax.dot_general(m2_ref[...], x2, (((1,), (0,)), ((), ())),
                             preferred_element_type=jnp.float32)
    c2 = jnp.maximum(c2.reshape(16, 10, 20, tb) + b3_ref[...], 0.0)

    a2 = jnp.max(c2.reshape(16, 5, 2, 20, tb), axis=2)
    sh2 = jnp.concatenate([a2[:, :, 2:, :], a2[:, :, :2, :]], axis=2)
    p2 = jnp.maximum(a2, sh2)
    feat = jnp.concatenate(
        [p2, jnp.zeros((16, 5, 4, tb), p2.dtype)], axis=2).reshape(1920, tb)

    h1 = jnp.maximum(
        jnp.dot(w5_ref[...], feat, preferred_element_type=jnp.float32)
        + b5_ref[...], 0.0)
    h2 = jnp.maximum(
        jnp.dot(w6_ref[...], h1, preferred_element_type=jnp.float32)
        + b6_ref[...], 0.0)
    o_ref[...] = (jnp.dot(w7_ref[...], h2, preferred_element_type=jnp.float32)
                  + b7_ref[...])


def _band1(w1):
    h = jnp.arange(28)
    h2 = jnp.arange(32)
    d = h2[None, :] - h[:, None]
    mask = (d >= 0) & (d < 5)
    dc = jnp.clip(d, 0, 4)
    w1s = w1[:, 0]
    band = w1s[:, dc, :] * mask[None, :, :, None].astype(w1.dtype)
    return band.transpose(0, 1, 3, 2).reshape(168, 160)


def _band2(w3):
    oh = jnp.arange(10)
    h2 = jnp.arange(14)
    d = h2[None, :] - oh[:, None]
    mask = (d >= 0) & (d < 5)
    dc = jnp.clip(d, 0, 4)
    band = w3[:, :, dc, :] * mask[None, None, :, :, None].astype(w3.dtype)
    return band.transpose(0, 2, 4, 1, 3).reshape(160, 420)


def kernel(x, w1, b1, w3, b3, w5, b5, w6, b6, w7, b7):
    B = x.shape[0]
    Bp = _round_up(B, TB)

    xt = jnp.pad(x.reshape(B, 28, 28),
                 ((0, Bp - B), (2, 2), (2, 2))).transpose(1, 2, 0)

    m1 = _band1(w1)
    m2 = _band2(w3)
    b1b = jnp.broadcast_to(b1.reshape(6, 1, 1, 1), (6, 1, 1, TB))
    b3b = jnp.broadcast_to(b3.reshape(16, 1, 1, 1), (16, 1, 1, TB))

    w5p = jnp.zeros((120, 16, 5, 24), w5.dtype).at[:, :, :, 0:20:4].set(
        w5.reshape(120, 16, 5, 5)).reshape(120, 1920)
    b5b = jnp.broadcast_to(b5[:, None], (120, TB))
    w6p = jnp.pad(w6, ((0, 12), (0, 0)))
    b6b = jnp.broadcast_to(jnp.pad(b6, (0, 12))[:, None], (96, TB))
    w7p = jnp.pad(w7, ((0, 6), (0, 12)))
    b7b = jnp.broadcast_to(jnp.pad(b7, (0, 6))[:, None], (16, TB))

    out = pl.pallas_call(
        _lenet_kernel,
        out_shape=jax.ShapeDtypeStruct((16, Bp), jnp.float32),
        grid=(Bp // TB,),
        in_specs=[
            pl.BlockSpec((32, 32, TB), lambda i: (0, 0, i)),
            pl.BlockSpec((168, 160), lambda i: (0, 0)),
            pl.BlockSpec((6, 1, 1, TB), lambda i: (0, 0, 0, 0)),
            pl.BlockSpec((160, 420), lambda i: (0, 0)),
            pl.BlockSpec((16, 1, 1, TB), lambda i: (0, 0, 0, 0)),
            pl.BlockSpec((120, 1920), lambda i: (0, 0)),
            pl.BlockSpec((120, TB), lambda i: (0, 0)),
            pl.BlockSpec((96, 120), lambda i: (0, 0)),
            pl.BlockSpec((96, TB), lambda i: (0, 0)),
            pl.BlockSpec((16, 96), lambda i: (0, 0)),
            pl.BlockSpec((16, TB), lambda i: (0, 0)),
        ],
        out_specs=pl.BlockSpec((16, TB), lambda i: (0, i)),
        compiler_params=pltpu.CompilerParams(
            dimension_semantics=("parallel",)),
    )(xt, m1, b1b, m2, b3b, w5p, b5b, w6p, b6b, w7p, b7b)
    return out[:10, :B].T


# speedup vs baseline: 1.0307x; 1.0307x over previous
import functools

import jax
import jax.numpy as jnp
from jax.experimental import pallas as pl
from jax.experimental.pallas import tpu as pltpu

TB = 256


def _round_up(x, m):
    return (x + m - 1) // m * m


def _lenet_kernel(x_ref, m1_ref, b1_ref, m2_ref, b3_ref, w5_ref,
                  b5_ref, w6_ref, b6_ref, w7_ref, b7_ref, o_ref):
    tb = o_ref.shape[1]
    xp = x_ref[...]

    xcat = jnp.concatenate(
        [xp[:, j:j + 28, :] for j in range(5)], axis=0).astype(jnp.bfloat16)
    c1 = jax.lax.dot_general(m1_ref[...], xcat, (((1,), (0,)), ((), ())),
                             preferred_element_type=jnp.float32)
    c1 = jnp.maximum(c1.reshape(6, 28, 28, tb) + b1_ref[...], 0.0)

    ph = jnp.max(c1.reshape(6, 14, 2, 28, tb), axis=2)
    shw = jnp.concatenate([ph[:, :, 1:, :], ph[:, :, :1, :]], axis=2)
    p1 = jnp.maximum(ph, shw)

    x2 = jnp.concatenate(
        [p1[:, :, 2 * j:2 * j + 20, :].reshape(84, 20, tb) for j in range(5)],
        axis=0).astype(jnp.bfloat16)
    c2 = jax.lax.dot_general(m2_ref[...], x2, (((1,), (0,)), ((), ())),
                             preferred_element_type=jnp.float32)
    c2 = jnp.maximum(c2.reshape(16, 10, 20, tb) + b3_ref[...], 0.0)

    a2 = jnp.max(c2.reshape(16, 5, 2, 20, tb), axis=2)
    sh2 = jnp.concatenate([a2[:, :, 2:, :], a2[:, :, :2, :]], axis=2)
    p2 = jnp.maximum(a2, sh2)
    feat = jnp.concatenate(
        [p2, jnp.zeros((16, 5, 4, tb), p2.dtype)], axis=2).reshape(1920, tb)

    h1 = jnp.maximum(
        jnp.dot(w5_ref[...], feat, preferred_element_type=jnp.float32)
        + b5_ref[...], 0.0)
    h2 = jnp.maximum(
        jnp.dot(w6_ref[...], h1, preferred_element_type=jnp.float32)
        + b6_ref[...], 0.0)
    o_ref[...] = (jnp.dot(w7_ref[...], h2, preferred_element_type=jnp.float32)
                  + b7_ref[...])


def _band1(w1):
    h = jnp.arange(28)
    h2 = jnp.arange(32)
    d = h2[None, :] - h[:, None]
    mask = (d >= 0) & (d < 5)
    dc = jnp.clip(d, 0, 4)
    w1s = w1[:, 0]
    band = w1s[:, dc, :] * mask[None, :, :, None].astype(w1.dtype)
    return band.transpose(0, 1, 3, 2).reshape(168, 160)


def _band2(w3):
    oh = jnp.arange(10)
    h2 = jnp.arange(14)
    d = h2[None, :] - oh[:, None]
    mask = (d >= 0) & (d < 5)
    dc = jnp.clip(d, 0, 4)
    band = w3[:, :, dc, :] * mask[None, None, :, :, None].astype(w3.dtype)
    return band.transpose(0, 2, 4, 1, 3).reshape(160, 420)


def kernel(x, w1, b1, w3, b3, w5, b5, w6, b6, w7, b7):
    B = x.shape[0]
    Bp = _round_up(B, TB)

    xt = jnp.pad(x.reshape(B, 28, 28),
                 ((0, Bp - B), (2, 2), (2, 2))).transpose(1, 2, 0)

    m1 = _band1(w1).astype(jnp.bfloat16)
    m2 = _band2(w3).astype(jnp.bfloat16)
    b1b = jnp.broadcast_to(b1.reshape(6, 1, 1, 1), (6, 1, 1, TB))
    b3b = jnp.broadcast_to(b3.reshape(16, 1, 1, 1), (16, 1, 1, TB))

    w5p = jnp.zeros((120, 16, 5, 24), w5.dtype).at[:, :, :, 0:20:4].set(
        w5.reshape(120, 16, 5, 5)).reshape(120, 1920)
    b5b = jnp.broadcast_to(b5[:, None], (120, TB))
    w6p = jnp.pad(w6, ((0, 12), (0, 0)))
    b6b = jnp.broadcast_to(jnp.pad(b6, (0, 12))[:, None], (96, TB))
    w7p = jnp.pad(w7, ((0, 6), (0, 12)))
    b7b = jnp.broadcast_to(jnp.pad(b7, (0, 6))[:, None], (16, TB))

    out = pl.pallas_call(
        _lenet_kernel,
        out_shape=jax.ShapeDtypeStruct((16, Bp), jnp.float32),
        grid=(Bp // TB,),
        in_specs=[
            pl.BlockSpec((32, 32, TB), lambda i: (0, 0, i)),
            pl.BlockSpec((168, 160), lambda i: (0, 0)),
            pl.BlockSpec((6, 1, 1, TB), lambda i: (0, 0, 0, 0)),
            pl.BlockSpec((160, 420), lambda i: (0, 0)),
            pl.BlockSpec((16, 1, 1, TB), lambda i: (0, 0, 0, 0)),
            pl.BlockSpec((120, 1920), lambda i: (0, 0)),
            pl.BlockSpec((120, TB), lambda i: (0, 0)),
            pl.BlockSpec((96, 120), lambda i: (0, 0)),
            pl.BlockSpec((96, TB), lambda i: (0, 0)),
            pl.BlockSpec((16, 96), lambda i: (0, 0)),
            pl.BlockSpec((16, TB), lambda i: (0, 0)),
        ],
        out_specs=pl.BlockSpec((16, TB), lambda i: (0, i)),
        compiler_params=pltpu.CompilerParams(
            dimension_semantics=("parallel",)),
    )(xt, m1, b1b, m2, b3b, w5p, b5b, w6p, b6b, w7p, b7b)
    return out[:10, :B].T


# speedup vs baseline: 1.0352x; 1.0044x over previous
import functools

import jax
import jax.numpy as jnp
from jax.experimental import pallas as pl
from jax.experimental.pallas import tpu as pltpu

TB = 256


def _round_up(x, m):
    return (x + m - 1) // m * m


def _half(xp, m1, b1, m2, b3, w5, b5, w6, b6, w7, b7):
    tb = xp.shape[2]

    xcat = jnp.concatenate(
        [xp[:, j:j + 28, :] for j in range(5)], axis=0).astype(jnp.bfloat16)
    c1 = jax.lax.dot_general(m1, xcat, (((1,), (0,)), ((), ())),
                             preferred_element_type=jnp.float32)
    c1 = jnp.maximum(c1.reshape(6, 28, 28, tb) + b1, 0.0)

    ph = jnp.max(c1.reshape(6, 14, 2, 28, tb), axis=2)
    shw = jnp.concatenate([ph[:, :, 1:, :], ph[:, :, :1, :]], axis=2)
    p1 = jnp.maximum(ph, shw)

    x2 = jnp.concatenate(
        [p1[:, :, 2 * j:2 * j + 20, :].reshape(84, 20, tb) for j in range(5)],
        axis=0).astype(jnp.bfloat16)
    c2 = jax.lax.dot_general(m2, x2, (((1,), (0,)), ((), ())),
                             preferred_element_type=jnp.float32)
    c2 = jnp.maximum(c2.reshape(16, 10, 20, tb) + b3, 0.0)

    a2 = jnp.max(c2.reshape(16, 5, 2, 20, tb), axis=2)
    sh2 = jnp.concatenate([a2[:, :, 2:, :], a2[:, :, :2, :]], axis=2)
    p2 = jnp.maximum(a2, sh2)
    feat = jnp.concatenate(
        [p2, jnp.zeros((16, 5, 4, tb), p2.dtype)], axis=2).reshape(1920, tb)

    h1 = jnp.maximum(
        jnp.dot(w5, feat, preferred_element_type=jnp.float32) + b5, 0.0)
    h2 = jnp.maximum(
        jnp.dot(w6, h1, preferred_element_type=jnp.float32) + b6, 0.0)
    return jnp.dot(w7, h2, preferred_element_type=jnp.float32) + b7


def _lenet_kernel(x_ref, m1_ref, b1_ref, m2_ref, b3_ref, w5_ref,
                  b5_ref, w6_ref, b6_ref, w7_ref, b7_ref, o_ref):
    tb = o_ref.shape[1]
    xp = x_ref[...]
    for g in range(tb // 128):
        s = slice(g * 128, (g + 1) * 128)
        o_ref[:, s] = _half(
            xp[:, :, s], m1_ref[...], b1_ref[..., s], m2_ref[...],
            b3_ref[..., s], w5_ref[...], b5_ref[:, s], w6_ref[...],
            b6_ref[:, s], w7_ref[...], b7_ref[:, s])


def _band1(w1):
    h = jnp.arange(28)
    h2 = jnp.arange(32)
    d = h2[None, :] - h[:, None]
    mask = (d >= 0) & (d < 5)
    dc = jnp.clip(d, 0, 4)
    w1s = w1[:, 0]
    band = w1s[:, dc, :] * mask[None, :, :, None].astype(w1.dtype)
    return band.transpose(0, 1, 3, 2).reshape(168, 160)


def _band2(w3):
    oh = jnp.arange(10)
    h2 = jnp.arange(14)
    d = h2[None, :] - oh[:, None]
    mask = (d >= 0) & (d < 5)
    dc = jnp.clip(d, 0, 4)
    band = w3[:, :, dc, :] * mask[None, None, :, :, None].astype(w3.dtype)
    return band.transpose(0, 2, 4, 1, 3).reshape(160, 420)


def kernel(x, w1, b1, w3, b3, w5, b5, w6, b6, w7, b7):
    B = x.shape[0]
    Bp = _round_up(B, TB)

    xt = jnp.pad(x.reshape(B, 28, 28),
                 ((0, Bp - B), (2, 2), (2, 2))).transpose(1, 2, 0)

    m1 = _band1(w1).astype(jnp.bfloat16)
    m2 = _band2(w3).astype(jnp.bfloat16)
    b1b = jnp.broadcast_to(b1.reshape(6, 1, 1, 1), (6, 1, 1, TB))
    b3b = jnp.broadcast_to(b3.reshape(16, 1, 1, 1), (16, 1, 1, TB))

    w5p = jnp.zeros((120, 16, 5, 24), w5.dtype).at[:, :, :, 0:20:4].set(
        w5.reshape(120, 16, 5, 5)).reshape(120, 1920)
    b5b = jnp.broadcast_to(b5[:, None], (120, TB))
    w6p = jnp.pad(w6, ((0, 12), (0, 0)))
    b6b = jnp.broadcast_to(jnp.pad(b6, (0, 12))[:, None], (96, TB))
    w7p = jnp.pad(w7, ((0, 6), (0, 12)))
    b7b = jnp.broadcast_to(jnp.pad(b7, (0, 6))[:, None], (16, TB))

    out = pl.pallas_call(
        _lenet_kernel,
        out_shape=jax.ShapeDtypeStruct((16, Bp), jnp.float32),
        grid=(Bp // TB,),
        in_specs=[
            pl.BlockSpec((32, 32, TB), lambda i: (0, 0, i)),
            pl.BlockSpec((168, 160), lambda i: (0, 0)),
            pl.BlockSpec((6, 1, 1, TB), lambda i: (0, 0, 0, 0)),
            pl.BlockSpec((160, 420), lambda i: (0, 0)),
            pl.BlockSpec((16, 1, 1, TB), lambda i: (0, 0, 0, 0)),
            pl.BlockSpec((120, 1920), lambda i: (0, 0)),
            pl.BlockSpec((120, TB), lambda i: (0, 0)),
            pl.BlockSpec((96, 120), lambda i: (0, 0)),
            pl.BlockSpec((96, TB), lambda i: (0, 0)),
            pl.BlockSpec((16, 96), lambda i: (0, 0)),
            pl.BlockSpec((16, TB), lambda i: (0, 0)),
        ],
        out_specs=pl.BlockSpec((16, TB), lambda i: (0, i)),
        compiler_params=pltpu.CompilerParams(
            dimension_semantics=("parallel",)),
    )(xt, m1, b1b, m2, b3b, w5p, b5b, w6p, b6b, w7p, b7b)
    return out[:10, :B].T


# speedup vs baseline: 1.0985x; 1.0611x over previous
import functools

import jax
import jax.numpy as jnp
from jax.experimental import pallas as pl
from jax.experimental.pallas import tpu as pltpu

TB = 256


def _round_up(x, m):
    return (x + m - 1) // m * m


def _half(xp, m1, m2, w5, w6, b6, w7, b7):
    tb = xp.shape[2]

    one = jnp.ones((1, 28, tb), jnp.bfloat16)
    xcat = jnp.concatenate(
        [xp[:, j:j + 28, :] for j in range(5)] + [one],
        axis=0).astype(jnp.bfloat16)
    c1 = jax.lax.dot_general(m1, xcat, (((1,), (0,)), ((), ())),
                             preferred_element_type=jnp.float32)
    c1 = jnp.maximum(c1.reshape(6, 28, 28, tb), 0.0)

    ph = jnp.max(c1.reshape(6, 14, 2, 28, tb), axis=2)
    shw = jnp.concatenate([ph[:, :, 1:, :], ph[:, :, :1, :]], axis=2)
    p1 = jnp.maximum(ph, shw)

    one2 = jnp.ones((1, 20, tb), jnp.bfloat16)
    x2 = jnp.concatenate(
        [p1[:, :, 2 * j:2 * j + 20, :].reshape(84, 20, tb).astype(jnp.bfloat16)
         for j in range(5)] + [one2], axis=0)
    c2 = jax.lax.dot_general(m2, x2, (((1,), (0,)), ((), ())),
                             preferred_element_type=jnp.float32)
    c2 = jnp.maximum(c2.reshape(16, 10, 20, tb), 0.0)

    a2 = jnp.max(c2.reshape(16, 5, 2, 20, tb), axis=2)
    sh2 = jnp.concatenate([a2[:, :, 2:, :], a2[:, :, :2, :]], axis=2)
    p2 = jnp.maximum(a2, sh2)
    feat = jnp.concatenate(
        [p2, jnp.zeros((16, 5, 3, tb), p2.dtype),
         jnp.ones((16, 5, 1, tb), p2.dtype)], axis=2).reshape(1920, tb)

    h1 = jnp.maximum(
        jnp.dot(w5, feat, preferred_element_type=jnp.float32), 0.0)
    h2 = jnp.maximum(
        jnp.dot(w6, h1, preferred_element_type=jnp.float32) + b6, 0.0)
    return jnp.dot(w7, h2, preferred_element_type=jnp.float32) + b7


def _lenet_kernel(x_ref, m1_ref, m2_ref, w5_ref,
                  w6_ref, b6_ref, w7_ref, b7_ref, o_ref):
    tb = o_ref.shape[1]
    xp = x_ref[...]
    for g in range(tb // 128):
        s = slice(g * 128, (g + 1) * 128)
        o_ref[:, s] = _half(
            xp[:, :, s], m1_ref[...], m2_ref[...], w5_ref[...],
            w6_ref[...], b6_ref[:, s], w7_ref[...], b7_ref[:, s])


def _band1(w1):
    h = jnp.arange(28)
    h2 = jnp.arange(32)
    d = h2[None, :] - h[:, None]
    mask = (d >= 0) & (d < 5)
    dc = jnp.clip(d, 0, 4)
    w1s = w1[:, 0]
    band = w1s[:, dc, :] * mask[None, :, :, None].astype(w1.dtype)
    return band.transpose(0, 1, 3, 2).reshape(168, 160)


def _band2(w3):
    oh = jnp.arange(10)
    h2 = jnp.arange(14)
    d = h2[None, :] - oh[:, None]
    mask = (d >= 0) & (d < 5)
    dc = jnp.clip(d, 0, 4)
    band = w3[:, :, dc, :] * mask[None, None, :, :, None].astype(w3.dtype)
    return band.transpose(0, 2, 4, 1, 3).reshape(160, 420)


def kernel(x, w1, b1, w3, b3, w5, b5, w6, b6, w7, b7):
    B = x.shape[0]
    Bp = _round_up(B, TB)

    xt = jnp.pad(x.reshape(B, 28, 28),
                 ((0, Bp - B), (2, 2), (2, 2))).transpose(1, 2, 0)

    m1 = jnp.concatenate(
        [_band1(w1), jnp.repeat(b1, 28)[:, None]],
        axis=1).astype(jnp.bfloat16)
    m2 = jnp.concatenate(
        [_band2(w3), jnp.repeat(b3, 10)[:, None]],
        axis=1).astype(jnp.bfloat16)

    w5p = jnp.zeros((120, 16, 5, 24), w5.dtype).at[:, :, :, 0:20:4].set(
        w5.reshape(120, 16, 5, 5)).at[:, 0, 0, 23].set(b5).reshape(120, 1920)
    w6p = jnp.pad(w6, ((0, 12), (0, 0)))
    b6b = jnp.broadcast_to(jnp.pad(b6, (0, 12))[:, None], (96, TB))
    w7p = jnp.pad(w7, ((0, 6), (0, 12)))
    b7b = jnp.broadcast_to(jnp.pad(b7, (0, 6))[:, None], (16, TB))

    out = pl.pallas_call(
        _lenet_kernel,
        out_shape=jax.ShapeDtypeStruct((16, Bp), jnp.float32),
        grid=(Bp // TB,),
        in_specs=[
            pl.BlockSpec((32, 32, TB), lambda i: (0, 0, i)),
            pl.BlockSpec((168, 161), lambda i: (0, 0)),
            pl.BlockSpec((160, 421), lambda i: (0, 0)),
            pl.BlockSpec((120, 1920), lambda i: (0, 0)),
            pl.BlockSpec((96, 120), lambda i: (0, 0)),
            pl.BlockSpec((96, TB), lambda i: (0, 0)),
            pl.BlockSpec((16, 96), lambda i: (0, 0)),
            pl.BlockSpec((16, TB), lambda i: (0, 0)),
        ],
        out_specs=pl.BlockSpec((16, TB), lambda i: (0, i)),
        compiler_params=pltpu.CompilerParams(
            dimension_semantics=("parallel",)),
    )(xt, m1, m2, w5p, w6p, b6b, w7p, b7b)
    return out[:10, :B].T
